# baseline (speedup 1.0000x reference)
import functools

import jax
import jax.numpy as jnp
from jax.experimental import pallas as pl
from jax.experimental.pallas import tpu as pltpu

_N, _C, _H, _W = 2, 96, 224, 224
_CM = 256
_KEEP = 4096
_TH = 16
_T = _H // _TH
_XR = _TH + 16
_R1, _W1C = _TH + 6, _W + 6
_R2, _W2C = _TH + 2, _W + 2


def _fused_body(par_ref, xp_ref, w1_ref, w2_ref, w3_ref, b1_ref, b2_ref,
                gn_ref, gs_ref, gh_ref, s0_ref, xbuf, h1buf, h2buf, sem):
    n = pl.program_id(0)
    t = pl.program_id(1)
    r0 = t * _TH

    cp = pltpu.make_async_copy(
        xp_ref.at[n, :, pl.ds(r0, _XR), :], xbuf, sem)
    cp.start()
    cp.wait()

    acc1 = jnp.zeros((_CM, _R1 * _W1C), jnp.float32)
    for ky in range(3):
        for kx in range(3):
            rhs = xbuf[:, 3 * ky:3 * ky + _R1, 3 * kx:3 * kx + _W1C]
            rhs = rhs.reshape(_C, _R1 * _W1C)
            acc1 = acc1 + jnp.dot(w1_ref[3 * ky + kx], rhs,
                                  preferred_element_type=jnp.float32)
    acc1 = acc1.reshape(_CM, _R1, _W1C) + b1_ref[...][:, :, None]
    rows = jax.lax.broadcasted_iota(jnp.int32, (_R1, _W1C), 0) + (r0 - 3)
    cols = jax.lax.broadcasted_iota(jnp.int32, (_R1, _W1C), 1) - 3
    valid = (rows >= 0) & (rows < _H) & (cols >= 0) & (cols < _W)
    h1buf[...] = jnp.where(valid[None], jnp.maximum(acc1, 0.0), 0.0)

    acc2 = jnp.zeros((_CM, _R2 * _W2C), jnp.float32)
    for ky in range(3):
        for kx in range(3):
            rhs = h1buf[:, 2 * ky:2 * ky + _R2, 2 * kx:2 * kx + _W2C]
            rhs = rhs.reshape(_CM, _R2 * _W2C)
            acc2 = acc2 + jnp.dot(w2_ref[3 * ky + kx], rhs,
                                  preferred_element_type=jnp.float32)
    acc2 = acc2.reshape(_CM, _R2, _W2C) + b2_ref[...][:, :, None]
    rows = jax.lax.broadcasted_iota(jnp.int32, (_R2, _W2C), 0) + (r0 - 1)
    cols = jax.lax.broadcasted_iota(jnp.int32, (_R2, _W2C), 1) - 1
    valid = (rows >= 0) & (rows < _H) & (cols >= 0) & (cols < _W)
    h2buf[...] = jnp.where(valid[None], jnp.maximum(acc2, 0.0), 0.0)

    out9 = jnp.dot(w3_ref[...], h2buf[...].reshape(_CM, _R2 * _W2C),
                   preferred_element_type=jnp.float32)
    out9 = out9.reshape(16, _R2, _W2C)
    h3 = jnp.zeros((_TH, _W), jnp.float32)
    for ky in range(3):
        for kx in range(3):
            h3 = h3 + out9[3 * ky + kx, ky:ky + _TH, kx:kx + _W]

    tau = par_ref[0, 0]
    b3 = par_ref[0, 1]
    h3 = h3 + b3
    p = 1.0 / (1.0 + jnp.exp(-h3))
    s0 = p + 1e-5
    l0 = jnp.log(s0)
    l1 = jnp.log((1.0 - p) + 1e-5)
    a0 = (l0 + gn_ref[0, 0]) / tau
    a1 = (l1 + gn_ref[0, 1]) / tau
    m = jnp.maximum(a0, a1)
    e0 = jnp.exp(a0 - m)
    e1 = jnp.exp(a1 - m)
    gs0 = e0 / (e0 + e1)
    hard = (a0 >= a1).astype(jnp.float32)
    gs_ref[0] = gs0
    gh_ref[0] = (hard - gs0) + gs0
    s0_ref[0] = s0


def _topk_body(gs_ref, mask_ref):
    bits = jax.lax.bitcast_convert_type(gs_ref[0], jnp.int32)
    rows = jax.lax.broadcasted_iota(jnp.int32, bits.shape, 0)
    cols = jax.lax.broadcasted_iota(jnp.int32, bits.shape, 1)
    idx = rows * bits.shape[1] + cols

    def vstep(i, v):
        cand = v | (jnp.int32(1) << (30 - i))
        cnt = jnp.sum((bits >= cand).astype(jnp.int32))
        return jnp.where(cnt >= _KEEP, cand, v)

    thr = jax.lax.fori_loop(0, 31, vstep, jnp.int32(0))
    gt = bits > thr
    eq = bits == thr
    e = _KEEP - jnp.sum(gt.astype(jnp.int32))

    def istep(i, lo):
        cand = lo | (jnp.int32(1) << (15 - i))
        cnt = jnp.sum((eq & (idx < cand)).astype(jnp.int32))
        return jnp.where(cnt < e, cand, lo)

    tie = jax.lax.fori_loop(0, 16, istep, jnp.int32(0))
    mask_ref[0] = (gt | (eq & (idx <= tie))).astype(jnp.int32)


@jax.jit
def kernel(x, W1, g1, b1, W2, g2, b2, W3, b3, tau):
    f32 = jnp.float32
    xp = jnp.pad(x, ((0, 0), (0, 0), (6, 10), (6, 6)))
    w1r = jnp.transpose(W1 * g1[:, None, None, None],
                        (2, 3, 0, 1)).reshape(9, _CM, _C)
    w2r = jnp.transpose(W2 * g2[:, None, None, None],
                        (2, 3, 0, 1)).reshape(9, _CM, _CM)
    w3r = jnp.zeros((16, _CM), f32).at[:9].set(
        jnp.transpose(W3, (2, 3, 0, 1)).reshape(9, _CM))
    u = jax.random.uniform(jax.random.key(42), (_N, 2, _H, _W),
                           minval=1e-10, maxval=1.0)
    gn = -jnp.log(-jnp.log(u))
    par = jnp.stack([jnp.asarray(tau, f32),
                     b3[0].astype(f32)]).reshape(1, 2)
    b1r = b1.reshape(_CM, 1)
    b2r = b2.reshape(_CM, 1)

    gs, gh, s0 = pl.pallas_call(
        _fused_body,
        grid=(_N, _T),
        in_specs=[
            pl.BlockSpec(memory_space=pltpu.SMEM),
            pl.BlockSpec(memory_space=pl.ANY),
            pl.BlockSpec((9, _CM, _C), lambda n, t: (0, 0, 0)),
            pl.BlockSpec((9, _CM, _CM), lambda n, t: (0, 0, 0)),
            pl.BlockSpec((16, _CM), lambda n, t: (0, 0)),
            pl.BlockSpec((_CM, 1), lambda n, t: (0, 0)),
            pl.BlockSpec((_CM, 1), lambda n, t: (0, 0)),
            pl.BlockSpec((1, 2, _TH, _W), lambda n, t: (n, 0, t, 0)),
        ],
        out_specs=[
            pl.BlockSpec((1, _TH, _W), lambda n, t: (n, t, 0)),
            pl.BlockSpec((1, _TH, _W), lambda n, t: (n, t, 0)),
            pl.BlockSpec((1, _TH, _W), lambda n, t: (n, t, 0)),
        ],
        out_shape=[jax.ShapeDtypeStruct((_N, _H, _W), f32)] * 3,
        scratch_shapes=[
            pltpu.VMEM((_C, _XR, _W + 12), f32),
            pltpu.VMEM((_CM, _R1, _W1C), f32),
            pltpu.VMEM((_CM, _R2, _W2C), f32),
            pltpu.SemaphoreType.DMA,
        ],
    )(par, xp, w1r, w2r, w3r, b1r, b2r, gn)

    gsr = gs.reshape(_N, _H * _W // 128, 128)
    maski = pl.pallas_call(
        _topk_body,
        grid=(_N,),
        in_specs=[pl.BlockSpec((1,) + gsr.shape[1:], lambda n: (n, 0, 0))],
        out_specs=pl.BlockSpec((1,) + gsr.shape[1:], lambda n: (n, 0, 0)),
        out_shape=jax.ShapeDtypeStruct(gsr.shape, jnp.int32),
    )(gsr)

    topk_mask = maski.reshape(_N, _H * _W).astype(bool)
    return topk_mask, gh.reshape(_N, _H * _W), s0


# speedup vs baseline: 3.0808x; 3.0808x over previous
import functools

import jax
import jax.numpy as jnp
from jax.experimental import pallas as pl
from jax.experimental.pallas import tpu as pltpu
from jax.experimental.pallas import tpu_sc as plsc
from jax import lax

_N, _C, _H, _W = 2, 96, 224, 224
_CM = 256
_KEEP = 4096
_TH = 32
_T = _H // _TH
_XR = _TH + 13
_WP = 256
_R1 = _TH + 6
_R2 = _TH + 2
_S1 = _R1 * _WP
_S2 = _R2 * _WP
_S3 = _TH * _WP
_S1P = _S1 + 128
_S2P = _S2 + 128
_CATW = 2 * 3 * _WP + _S1


def _fused_body(par_ref, xp_ref, w1_ref, w2_ref, w3_ref, b1_ref, b2_ref,
                gn_ref, gs_ref, gh_ref, s0_ref, xbuf, cat3, h1buf, h2buf,
                sem):
    n = pl.program_id(0)
    t = pl.program_id(1)
    r0 = t * _TH
    slot = jax.lax.rem(t, 2)

    def xdma(tt, sl):
        return pltpu.make_async_copy(
            xp_ref.at[n, :, pl.ds(tt * _TH * _WP, _XR * _WP)],
            xbuf.at[sl], sem.at[sl])

    @pl.when(t == 0)
    def _():
        xdma(t, slot).start()

    @pl.when(t + 1 < _T)
    def _():
        xdma(t + 1, 1 - slot).start()

    xdma(t, slot).wait()

    for kx in range(3):
        cat3[96 * kx:96 * (kx + 1), :] = xbuf[slot, :,
                                              3 * kx:3 * kx + _CATW]
    acc1 = jnp.zeros((_CM, _S1), jnp.float32)
    for ky in range(3):
        acc1 = acc1 + jnp.dot(w1_ref[ky],
                              cat3[:, 3 * ky * _WP:3 * ky * _WP + _S1],
                              preferred_element_type=jnp.float32)
    acc1 = acc1 + b1_ref[...]
    pos = jax.lax.broadcasted_iota(jnp.int32, (1, _S1), 1)
    row = (pos >> 8) + (r0 - 3)
    col = (pos & 255) - 3
    valid = (row >= 0) & (row < _H) & (col >= 0) & (col < _W)
    h1buf[:, :_S1] = jnp.where(valid, jnp.maximum(acc1, 0.0), 0.0)

    acc2 = jnp.zeros((_CM, _S2), jnp.float32)
    for ky in range(3):
        for kx in range(3):
            rhs = h1buf[:, 2 * ky * _WP + 2 * kx:2 * ky * _WP + 2 * kx + _S2]
            acc2 = acc2 + jnp.dot(w2_ref[3 * ky + kx], rhs,
                                  preferred_element_type=jnp.float32)
    acc2 = acc2 + b2_ref[...]
    pos = jax.lax.broadcasted_iota(jnp.int32, (1, _S2), 1)
    row = (pos >> 8) + (r0 - 1)
    col = (pos & 255) - 1
    valid = (row >= 0) & (row < _H) & (col >= 0) & (col < _W)
    h2buf[:, :_S2] = jnp.where(valid, jnp.maximum(acc2, 0.0), 0.0)
    h2buf[:, _S2:] = jnp.zeros((_CM, 128), jnp.float32)

    out9 = jnp.dot(w3_ref[...], h2buf[...],
                   preferred_element_type=jnp.float32)
    h3f = jnp.zeros((1, _S3), jnp.float32)
    for ky in range(3):
        for kx in range(3):
            tap = 3 * ky + kx
            off = ky * _WP + kx
            h3f = h3f + out9[tap:tap + 1, off:off + _S3]

    h3 = h3f.reshape(_TH, _WP)[:, :_W]
    tau = par_ref[0, 0]
    b3 = par_ref[0, 1]
    h3 = h3 + b3
    p = 1.0 / (1.0 + jnp.exp(-h3))
    s0 = p + 1e-5
    l0 = jnp.log(s0)
    l1 = jnp.log((1.0 - p) + 1e-5)
    a0 = (l0 + gn_ref[0, 0]) / tau
    a1 = (l1 + gn_ref[0, 1]) / tau
    m = jnp.maximum(a0, a1)
    e0 = jnp.exp(a0 - m)
    e1 = jnp.exp(a1 - m)
    gs0 = e0 / (e0 + e1)
    hard = (a0 >= a1).astype(jnp.float32)
    gs_ref[0] = jax.lax.bitcast_convert_type(gs0, jnp.int32)
    gh_ref[0] = (hard - gs0) + gs0
    s0_ref[0] = s0


def _topk_body(gs_ref, mask_ref):
    bits = gs_ref[0]
    rows = jax.lax.broadcasted_iota(jnp.int32, bits.shape, 0)
    cols = jax.lax.broadcasted_iota(jnp.int32, bits.shape, 1)
    idx = rows * bits.shape[1] + cols

    def vstep(i, v):
        cand = v | (jnp.int32(1) << (30 - i))
        cnt = jnp.sum((bits >= cand).astype(jnp.int32))
        return jnp.where(cnt >= _KEEP, cand, v)

    thr = jax.lax.fori_loop(0, 31, vstep, jnp.int32(0))
    gt = bits > thr
    eq = bits == thr
    e = _KEEP - jnp.sum(gt.astype(jnp.int32))

    def istep(i, lo):
        cand = lo | (jnp.int32(1) << (15 - i))
        cnt = jnp.sum((eq & (idx < cand)).astype(jnp.int32))
        return jnp.where(cnt < e, cand, lo)

    tie = jax.lax.fori_loop(0, 16, istep, jnp.int32(0))
    mask_ref[0] = (gt | (eq & (idx <= tie))).astype(jnp.int32)


_ROW = _H * _W
_NS = 16
_CHUNK = _ROW // _NS
_CHUNKP = 3200
_NV = _CHUNKP // 16

def _sc_topk(gs2):
    mesh = plsc.VectorSubcoreMesh(core_axis_name="c", subcore_axis_name="s")

    @functools.partial(
        pl.kernel,
        mesh=mesh,
        compiler_params=pltpu.CompilerParams(needs_layout_passes=False),
        out_type=jax.ShapeDtypeStruct((2, 1, _NS * _CHUNKP), jnp.int32),
        scratch_types=[
            pltpu.VMEM((_CHUNKP,), jnp.int32),
            pltpu.VMEM((_CHUNKP,), jnp.int32),
            pltpu.VMEM((16,), jnp.int32),
            pltpu.VMEM((_NS, 16), jnp.int32),
            pltpu.VMEM_SHARED((2, _NS, 16), jnp.int32),
        ],
    )
    def k(gs_hbm, mask_hbm, vals_v, mask_v, cnt_v, allc_v, shared):
        core = lax.axis_index("c")
        sid = lax.axis_index("s")
        base = sid * _CHUNK
        pbase = sid * _CHUNKP
        pltpu.sync_copy(gs_hbm.at[core, 0, pl.ds(pbase, _CHUNKP)], vals_v)

        lanes = lax.iota(jnp.int32, 16)
        one = jnp.ones((16,), jnp.int32)
        zero = jnp.zeros((16,), jnp.int32)

        def gcount(pred):
            def body(i, c16):
                b = vals_v[pl.ds(i * 16, 16)]
                idx = base + i * 16 + lanes
                return c16 + jnp.where(pred(b, idx), one, zero)

            cnt_v[...] = lax.fori_loop(0, _NV, body, zero)
            pltpu.sync_copy(cnt_v, shared.at[core, sid])
            plsc.subcore_barrier()
            pltpu.sync_copy(shared.at[core], allc_v)
            acc = jnp.zeros((16,), jnp.int32)
            for j in range(_NS):
                acc = acc + allc_v[j]
            plsc.subcore_barrier()
            return jnp.sum(acc)

        def vround(i, v):
            cand = v | (jnp.int32(1) << (jnp.int32(30) - i))
            cnt = gcount(lambda b, idx: b >= cand)
            return jnp.where(cnt >= _KEEP, cand, v)

        thr = lax.fori_loop(0, 31, vround, jnp.int32(0))
        m = gcount(lambda b, idx: b > thr)
        e = _KEEP - m

        def iround(i, lo):
            cand = lo | (jnp.int32(1) << (jnp.int32(15) - i))
            cnt = gcount(lambda b, idx: (b == thr) & (idx < cand))
            return jnp.where(cnt < e, cand, lo)

        tie = lax.fori_loop(0, 16, iround, jnp.int32(0))

        def wr(i, _):
            b = vals_v[pl.ds(i * 16, 16)]
            idx = base + i * 16 + lanes
            sel = (b > thr) | ((b == thr) & (idx <= tie))
            mask_v[pl.ds(i * 16, 16)] = jnp.where(sel, one, zero)
            return 0

        lax.fori_loop(0, _NV, wr, 0)
        pltpu.sync_copy(mask_v, mask_hbm.at[core, 0, pl.ds(pbase, _CHUNKP)])

    return k(gs2)


@jax.jit
def kernel(x, W1, g1, b1, W2, g2, b2, W3, b3, tau):
    f32 = jnp.float32
    xp = jnp.pad(x, ((0, 0), (0, 0), (6, 10), (6, 26)))
    xp = xp.reshape(_N, _C, 240 * _WP)
    w1r = jnp.transpose(W1 * g1[:, None, None, None],
                        (2, 0, 3, 1)).reshape(3, _CM, 3 * _C)
    w2r = jnp.transpose(W2 * g2[:, None, None, None],
                        (2, 3, 0, 1)).reshape(9, _CM, _CM)
    w3r = jnp.zeros((16, _CM), f32).at[:9].set(
        jnp.transpose(W3, (2, 3, 0, 1)).reshape(9, _CM))
    u = jax.random.uniform(jax.random.key(42), (_N, 2, _H, _W),
                           minval=1e-10, maxval=1.0)
    gn = -jnp.log(-jnp.log(u))
    par = jnp.stack([jnp.asarray(tau, f32),
                     b3[0].astype(f32)]).reshape(1, 2)
    b1r = b1.reshape(_CM, 1)
    b2r = b2.reshape(_CM, 1)

    gs, gh, s0 = pl.pallas_call(
        _fused_body,
        grid=(_N, _T),
        in_specs=[
            pl.BlockSpec(memory_space=pltpu.SMEM),
            pl.BlockSpec(memory_space=pl.ANY),
            pl.BlockSpec((3, _CM, 3 * _C), lambda n, t: (0, 0, 0)),
            pl.BlockSpec((9, _CM, _CM), lambda n, t: (0, 0, 0)),
            pl.BlockSpec((16, _CM), lambda n, t: (0, 0)),
            pl.BlockSpec((_CM, 1), lambda n, t: (0, 0)),
            pl.BlockSpec((_CM, 1), lambda n, t: (0, 0)),
            pl.BlockSpec((1, 2, _TH, _W), lambda n, t: (n, 0, t, 0)),
        ],
        out_specs=[
            pl.BlockSpec((1, _TH, _W), lambda n, t: (n, t, 0)),
            pl.BlockSpec((1, _TH, _W), lambda n, t: (n, t, 0)),
            pl.BlockSpec((1, _TH, _W), lambda n, t: (n, t, 0)),
        ],
        out_shape=[jax.ShapeDtypeStruct((_N, _H, _W), jnp.int32),
                   jax.ShapeDtypeStruct((_N, _H, _W), f32),
                   jax.ShapeDtypeStruct((_N, _H, _W), f32)],
        scratch_shapes=[
            pltpu.VMEM((2, _C, _XR * _WP), f32),
            pltpu.VMEM((3 * _C, _CATW), f32),
            pltpu.VMEM((_CM, _S1P), f32),
            pltpu.VMEM((_CM, _S2P), f32),
            pltpu.SemaphoreType.DMA((2,)),
        ],
    )(par, xp, w1r, w2r, w3r, b1r, b2r, gn)

    USE_SC = False
    if USE_SC:
        gsp = jnp.pad(gs.reshape(_N, _NS, _CHUNK),
                      ((0, 0), (0, 0), (0, _CHUNKP - _CHUNK)))
        maski = _sc_topk(gsp.reshape(_N, 1, _NS * _CHUNKP))
        topk_mask = maski.reshape(_N, _NS, _CHUNKP)[:, :, :_CHUNK].reshape(
            _N, _H * _W).astype(bool)
    else:
        gsr = gs.reshape(_N, _H * _W // 128, 128)
        maski = pl.pallas_call(
            _topk_body,
            grid=(_N,),
            in_specs=[pl.BlockSpec((1,) + gsr.shape[1:], lambda n: (n, 0, 0))],
            out_specs=pl.BlockSpec((1,) + gsr.shape[1:], lambda n: (n, 0, 0)),
            out_shape=jax.ShapeDtypeStruct(gsr.shape, jnp.int32),
        )(gsr)
        topk_mask = maski.reshape(_N, _H * _W).astype(bool)
    return topk_mask, gh.reshape(_N, _H * _W), s0
